# baseline (speedup 1.0000x reference)
import functools

import jax
import jax.numpy as jnp
from jax import lax
from jax.experimental import pallas as pl
from jax.experimental.pallas import tpu as pltpu
from jax.experimental.pallas import tpu_sc as plsc

NC, NS = 2, 16
NW = NC * NS
NCHUNK = 25
DC = 100000 // NCHUNK
G = 8


def _make_gather(num_rows_flat):
    jpw = num_rows_flat // NW
    assert num_rows_flat % NW == 0 and jpw % G == 0

    mesh = plsc.VectorSubcoreMesh(core_axis_name="c", subcore_axis_name="s")

    @functools.partial(
        pl.kernel,
        out_type=jax.ShapeDtypeStruct((num_rows_flat, DC), jnp.float32),
        mesh=mesh,
        scratch_types=[
            pltpu.VMEM((jpw,), jnp.int32),
            pltpu.VMEM((G, DC), jnp.float32),
            pltpu.SemaphoreType.DMA,
        ],
        compiler_params=pltpu.CompilerParams(use_tc_tiling_on_sc=False),
    )
    def gather(idx_hbm, table_hbm, out_hbm, idx_v, buf, sem):
        wid = lax.axis_index("s") * NC + lax.axis_index("c")
        base = wid * jpw
        pltpu.sync_copy(idx_hbm.at[pl.ds(base, jpw)], idx_v)

        def body(b, _):
            off = b * G
            pltpu.async_copy(
                table_hbm.at[idx_v.at[pl.ds(off, G)]], buf, sem
            ).wait()
            pltpu.sync_copy(buf, out_hbm.at[pl.ds(base + off, G)])
            return ()

        lax.fori_loop(0, jpw // G, body, ())

    return gather


def kernel(e, weight):
    b = e.shape[0]
    v, d = weight.shape
    eidx = (
        e.astype(jnp.int32)[:, None] * NCHUNK
        + jnp.arange(NCHUNK, dtype=jnp.int32)[None, :]
    ).reshape(-1)
    wflat = weight.reshape(v * NCHUNK, DC)
    out = _make_gather(b * NCHUNK)(eidx, wflat)
    return out.reshape(b, 1, d)


# speedup vs baseline: 1.0367x; 1.0367x over previous
import functools

import jax
import jax.numpy as jnp
from jax import lax
from jax.experimental import pallas as pl
from jax.experimental.pallas import tpu as pltpu
from jax.experimental.pallas import tpu_sc as plsc

NC, NS = 2, 16
NW = NC * NS
NCHUNK = 25
DC = 100000 // NCHUNK
G = 8


def _make_gather(num_rows_flat):
    jpw = num_rows_flat // NW
    assert num_rows_flat % NW == 0 and jpw % G == 0

    mesh = plsc.VectorSubcoreMesh(core_axis_name="c", subcore_axis_name="s")

    nb = jpw // G
    assert nb >= 3

    @functools.partial(
        pl.kernel,
        out_type=jax.ShapeDtypeStruct((num_rows_flat, DC), jnp.float32),
        mesh=mesh,
        scratch_types=[
            pltpu.VMEM((jpw,), jnp.int32),
            pltpu.VMEM((2, G, DC), jnp.float32),
            pltpu.SemaphoreType.DMA((2,)),
            pltpu.SemaphoreType.DMA((2,)),
        ],
        compiler_params=pltpu.CompilerParams(use_tc_tiling_on_sc=False),
    )
    def gather(idx_hbm, table_hbm, out_hbm, idx_v, bufs, gsem, ssem):
        wid = lax.axis_index("s") * NC + lax.axis_index("c")
        base = wid * jpw
        pltpu.sync_copy(idx_hbm.at[pl.ds(base, jpw)], idx_v)

        def gcopy(b, slot):
            return pltpu.make_async_copy(
                table_hbm.at[idx_v.at[pl.ds(b * G, G)]],
                bufs.at[slot],
                gsem.at[slot],
            )

        def scopy(b, slot):
            return pltpu.make_async_copy(
                bufs.at[slot],
                out_hbm.at[pl.ds(base + b * G, G)],
                ssem.at[slot],
            )

        gcopy(0, 0).start()
        gcopy(1, 1).start()
        gcopy(0, 0).wait()
        scopy(0, 0).start()

        def body(b, _):
            slot = lax.rem(b, 2)
            nslot = lax.rem(b + 1, 2)
            gcopy(b, slot).wait()
            scopy(b, slot).start()
            scopy(b - 1, nslot).wait()
            gcopy(b + 1, nslot).start()
            return ()

        lax.fori_loop(1, nb - 1, body, ())

        last = nb - 1
        lslot = lax.rem(last, 2)
        gcopy(last, lslot).wait()
        scopy(last, lslot).start()
        scopy(last - 1, lax.rem(last + 1, 2)).wait()
        scopy(last, lslot).wait()

    return gather


def kernel(e, weight):
    b = e.shape[0]
    v, d = weight.shape
    eidx = (
        e.astype(jnp.int32)[:, None] * NCHUNK
        + jnp.arange(NCHUNK, dtype=jnp.int32)[None, :]
    ).reshape(-1)
    wflat = weight.reshape(v * NCHUNK, DC)
    out = _make_gather(b * NCHUNK)(eidx, wflat)
    return out.reshape(b, 1, d)


# speedup vs baseline: 1.0385x; 1.0018x over previous
import functools

import jax
import jax.numpy as jnp
from jax import lax
from jax.experimental import pallas as pl
from jax.experimental.pallas import tpu as pltpu
from jax.experimental.pallas import tpu_sc as plsc

NC, NS = 2, 16
NW = NC * NS
NCH = 2
L = 16


def _make_gather(b, v, d):
    rpw = b // NW
    chk = d // NCH
    nb = rpw * NCH
    assert b % NW == 0 and d % NCH == 0 and rpw % L == 0 and nb >= 3

    mesh = plsc.VectorSubcoreMesh(core_axis_name="c", subcore_axis_name="s")

    @functools.partial(
        pl.kernel,
        out_type=jax.ShapeDtypeStruct((b, d), jnp.float32),
        mesh=mesh,
        scratch_types=[
            pltpu.VMEM((rpw,), jnp.int32),
            pltpu.VMEM((2, 1, chk), jnp.float32),
            pltpu.SemaphoreType.DMA((2,)),
            pltpu.SemaphoreType.DMA((2,)),
        ],
        compiler_params=pltpu.CompilerParams(
            use_tc_tiling_on_sc=False, needs_layout_passes=False
        ),
    )
    def gather(e_hbm, table_hbm, out_hbm, idx_v, bufs, gsem, ssem):
        wid = lax.axis_index("s") * NC + lax.axis_index("c")
        base = wid * rpw
        pltpu.sync_copy(e_hbm.at[pl.ds(base, rpw)], idx_v)
        lanes = lax.iota(jnp.int32, 16)

        def row_index(batch):
            r = batch // NCH
            grp = (r // L) * L
            vec = idx_v[pl.ds(grp, L)]
            return jnp.sum(jnp.where(lanes == (r % L), vec, 0))

        def gcopy(batch, slot):
            ridx = row_index(batch)
            c = batch % NCH
            return pltpu.make_async_copy(
                table_hbm.at[pl.ds(ridx, 1), pl.ds(c * chk, chk)],
                bufs.at[slot],
                gsem.at[slot],
            )

        def scopy(batch, slot):
            r = batch // NCH
            c = batch % NCH
            return pltpu.make_async_copy(
                bufs.at[slot],
                out_hbm.at[pl.ds(base + r, 1), pl.ds(c * chk, chk)],
                ssem.at[slot],
            )

        gcopy(0, 0).start()
        gcopy(1, 1).start()
        gcopy(0, 0).wait()
        scopy(0, 0).start()

        def body(bt, _):
            slot = lax.rem(bt, 2)
            nslot = lax.rem(bt + 1, 2)
            gcopy(bt, slot).wait()
            scopy(bt, slot).start()
            scopy(bt - 1, nslot).wait()
            gcopy(bt + 1, nslot).start()
            return ()

        lax.fori_loop(1, nb - 1, body, ())

        last = nb - 1
        lslot = lax.rem(last, 2)
        gcopy(last, lslot).wait()
        scopy(last, lslot).start()
        scopy(last - 1, lax.rem(last + 1, 2)).wait()
        scopy(last, lslot).wait()

    return gather


def kernel(e, weight):
    b = e.shape[0]
    v, d = weight.shape
    out = _make_gather(b, v, d)(e.astype(jnp.int32), weight)
    return out.reshape(b, 1, d)


# speedup vs baseline: 3.0662x; 2.9524x over previous
import functools

import jax
import jax.numpy as jnp
from jax import lax
from jax.experimental import pallas as pl
from jax.experimental.pallas import tpu as pltpu
from jax.experimental.pallas import tpu_sc as plsc

NC, NS = 2, 16
NW = NC * NS
LANE = 128
CW = 1408


def _make_sc_gather(b, v, d):
    rpw = b // NW
    dal = (d // LANE) * LANE
    nb = dal // CW
    assert b % NW == 0 and rpw % 8 == 0 and dal % CW == 0 and nb >= 3

    mesh = plsc.VectorSubcoreMesh(core_axis_name="c", subcore_axis_name="s")

    @functools.partial(
        pl.kernel,
        out_type=jax.ShapeDtypeStruct((b, d), jnp.float32),
        mesh=mesh,
        scratch_types=[
            pltpu.VMEM((rpw,), jnp.int32),
            pltpu.VMEM((2, rpw, CW), jnp.float32),
            pltpu.SemaphoreType.DMA((2,)),
            pltpu.SemaphoreType.DMA((2,)),
        ],
    )
    def gather(e_hbm, table_hbm, out_hbm, idx_v, bufs, gsem, ssem):
        wid = lax.axis_index("s") * NC + lax.axis_index("c")
        base = wid * rpw
        pltpu.sync_copy(e_hbm.at[pl.ds(base, rpw)], idx_v)

        def gcopy(c, slot):
            return pltpu.make_async_copy(
                table_hbm.at[idx_v, pl.ds(c * CW, CW)],
                bufs.at[slot],
                gsem.at[slot],
            )

        def scopy(c, slot):
            return pltpu.make_async_copy(
                bufs.at[slot],
                out_hbm.at[pl.ds(base, rpw), pl.ds(c * CW, CW)],
                ssem.at[slot],
            )

        gcopy(0, 0).start()
        gcopy(1, 1).start()
        gcopy(0, 0).wait()
        scopy(0, 0).start()

        def body(c, _):
            slot = lax.rem(c, 2)
            nslot = lax.rem(c + 1, 2)
            gcopy(c, slot).wait()
            scopy(c, slot).start()
            scopy(c - 1, nslot).wait()
            gcopy(c + 1, nslot).start()
            return ()

        lax.fori_loop(1, nb - 1, body, ())

        last = nb - 1
        lslot = lax.rem(last, 2)
        gcopy(last, lslot).wait()
        scopy(last, lslot).start()
        scopy(last - 1, lax.rem(last + 1, 2)).wait()
        scopy(last, lslot).wait()

    return gather


def _make_tc_tail(b, v, d):
    dal = (d // LANE) * LANE
    tailblk = dal // LANE

    def tail_kernel(e_ref, wtail_ref, _, o_ref):
        e = e_ref[:]
        onehot = (
            e[:, None] == lax.broadcasted_iota(jnp.int32, (b, v), 1)
        ).astype(jnp.float32)
        o_ref[...] = jnp.dot(
            onehot, wtail_ref[...], preferred_element_type=jnp.float32
        )

    return pl.pallas_call(
        tail_kernel,
        grid=(1,),
        out_shape=jax.ShapeDtypeStruct((b, d), jnp.float32),
        in_specs=[
            pl.BlockSpec((b,), lambda i: (0,)),
            pl.BlockSpec((v, LANE), lambda i: (0, tailblk)),
            pl.BlockSpec(memory_space=pl.ANY),
        ],
        out_specs=pl.BlockSpec((b, LANE), lambda i: (0, tailblk)),
        input_output_aliases={2: 0},
    )


def kernel(e, weight):
    b = e.shape[0]
    v, d = weight.shape
    ei = e.astype(jnp.int32)
    out = _make_sc_gather(b, v, d)(ei, weight)
    out = _make_tc_tail(b, v, d)(ei, weight, out)
    return out.reshape(b, 1, d)
